# baseline (speedup 1.0000x reference)
import jax
import jax.numpy as jnp
from jax import lax
from jax.experimental import pallas as pl
from jax.experimental.pallas import tpu as pltpu

VT = 2048


def _mlp_body(e_ref, w1_ref, b1_ref, w2_ref, b2_ref, out_ref, h_ref):
    @pl.when(pl.program_id(0) == 0)
    def _():
        e = e_ref[...].astype(jnp.bfloat16)
        w1 = w1_ref[...].astype(jnp.bfloat16)
        h = lax.dot_general(e, w1, (((1,), (1,)), ((), ())),
                            preferred_element_type=jnp.float32)
        h = h + b1_ref[...][None, :]
        h_ref[...] = jnp.maximum(h, 0.0).astype(jnp.bfloat16)

    w2 = w2_ref[...].astype(jnp.bfloat16)
    out = lax.dot_general(h_ref[...], w2, (((1,), (1,)), ((), ())),
                          preferred_element_type=jnp.float32)
    out_ref[...] = out + b2_ref[...][None, :]


def _mlp(e, W1, b1, W2, b2):
    B = e.shape[0]
    HID = W1.shape[0]
    VOCAB = W2.shape[0]
    grid = (pl.cdiv(VOCAB, VT),)
    return pl.pallas_call(
        _mlp_body,
        grid=grid,
        in_specs=[
            pl.BlockSpec((B, e.shape[1]), lambda i: (0, 0)),
            pl.BlockSpec((HID, W1.shape[1]), lambda i: (0, 0)),
            pl.BlockSpec((HID,), lambda i: (0,)),
            pl.BlockSpec((VT, HID), lambda i: (i, 0)),
            pl.BlockSpec((VT,), lambda i: (i,)),
        ],
        out_specs=pl.BlockSpec((B, VT), lambda i: (0, i)),
        out_shape=jax.ShapeDtypeStruct((B, VOCAB), jnp.float32),
        scratch_shapes=[pltpu.VMEM((B, HID), jnp.bfloat16)],
        compiler_params=pltpu.CompilerParams(
            dimension_semantics=("arbitrary",),
        ),
    )(e, W1, b1, W2, b2)


@jax.jit
def kernel(x, emb, W1, b1, W2, b2):
    e = jnp.take(emb, x.reshape(-1), axis=0)
    e = e.reshape(x.shape[0], -1)
    return _mlp(e, W1, b1, W2, b2)
